# baseline (speedup 1.0000x reference)
import functools

import jax
import jax.numpy as jnp
from jax import lax
from jax.experimental import pallas as pl
from jax.experimental.pallas import tpu as pltpu
from jax.experimental.pallas import tpu_sc as plsc

_BEAM = 32
_VOCAB = 100000
_MAXLEN = 200
_EOS = 2
_LANES = 16
_NCHUNK = _VOCAB // _LANES
_NC = 2
_NS = 16


_CAPL = 512
_CAP = _CAPL * _LANES
_U1 = 10
_U2 = 10


def _merge_topk(st_v, sti_v, v, idx):
    sv, si = plsc.sort_key_val(v, idx, descending=True)
    hi = st_v[pl.ds(0, _LANES)]
    hii = sti_v[pl.ds(0, _LANES)]
    lo = st_v[pl.ds(_LANES, _LANES)]
    loi = sti_v[pl.ds(_LANES, _LANES)]
    rv = lax.rev(sv, (0,))
    ri = lax.rev(si, (0,))
    m = lo >= rv
    u = jnp.where(m, lo, rv)
    ui = jnp.where(m, loi, ri)
    us, usi = plsc.sort_key_val(u, ui, descending=True)
    ru = lax.rev(us, (0,))
    rui = lax.rev(usi, (0,))
    m2 = hi >= ru
    a = jnp.where(m2, hi, ru)
    ai = jnp.where(m2, hii, rui)
    b = jnp.where(m2, ru, hi)
    bi = jnp.where(m2, rui, hii)
    hs, hsi = plsc.sort_key_val(a, ai, descending=True)
    ls, lsi = plsc.sort_key_val(b, bi, descending=True)
    st_v[pl.ds(0, _LANES)] = hs
    sti_v[pl.ds(0, _LANES)] = hsi
    st_v[pl.ds(_LANES, _LANES)] = ls
    sti_v[pl.ds(_LANES, _LANES)] = lsi


def _sc_body(logits_hbm, scores_hbm, tv_hbm, ti_hbm,
             row_v, cand_v, st_v, sti_v, sc_v):
    wid = lax.axis_index("s") * _NC + lax.axis_index("c")
    pltpu.sync_copy(logits_hbm.at[wid], row_v)

    neg = jnp.full((_LANES,), -jnp.inf, jnp.float32)

    @plsc.parallel_loop(
        0, _VOCAB, _LANES, unroll=_U1,
        carry=(neg, neg, jnp.zeros((_LANES,), jnp.float32)))
    def p1(i, carry):
        m1, m2, acc = carry
        v = row_v[pl.ds(i, _LANES)]
        acc = acc + jnp.exp(v)
        m2 = jnp.maximum(m2, jnp.minimum(m1, v))
        m1 = jnp.maximum(m1, v)
        return m1, m2, acc

    m1, m2, acc = p1
    thr = jnp.broadcast_to(jnp.min(m2), (_LANES,))

    lane_base = lax.iota(jnp.int32, _LANES) * _CAPL

    @plsc.parallel_loop(
        0, _VOCAB, _LANES, unroll=_U2,
        carry=jnp.zeros((_LANES,), jnp.int32))
    def p2(i, cnt):
        v = row_v[pl.ds(i, _LANES)]
        bvec = lax.iota(jnp.int32, _LANES) + i
        msk = v >= thr
        mok = jnp.logical_and(msk, cnt < _CAPL)
        plsc.store_scatter(cand_v, [lane_base + cnt], bvec, mask=mok)
        return cnt + jnp.where(msk, 1, 0)

    cnt = p2

    st_v[pl.ds(0, _LANES)] = neg
    st_v[pl.ds(_LANES, _LANES)] = neg
    zero_i = jnp.zeros((_LANES,), jnp.int32)
    sti_v[pl.ds(0, _LANES)] = zero_i
    sti_v[pl.ds(_LANES, _LANES)] = zero_i

    cnt = jnp.minimum(cnt, _CAPL)
    for lane in range(_LANES):
        cl = cnt[lane]

        def p3(j, carry, lane=lane, cl=cl):
            b0 = lane * _CAPL + j * _LANES
            valid = lax.iota(jnp.int32, _LANES) + j * _LANES < cl
            idx = jnp.where(valid, cand_v[pl.ds(b0, _LANES)], 0)
            v = plsc.load_gather(row_v, [idx])
            v = jnp.where(valid, v, -jnp.inf)
            _merge_topk(st_v, sti_v, v, idx)
            return carry

        lax.fori_loop(0, (cl + _LANES - 1) // _LANES, p3, 0)

    pltpu.sync_copy(scores_hbm.at[wid], sc_v)
    x = jnp.broadcast_to(jnp.sum(acc), (_LANES,))
    bits = plsc.bitcast(x, jnp.int32)
    e = jnp.right_shift(bits, 23) - 127
    m = plsc.bitcast((bits & 0x7FFFFF) | 0x3F800000, jnp.float32)
    half = m > 1.4142135
    m = jnp.where(half, m * 0.5, m)
    ef = e.astype(jnp.float32) + jnp.where(half, 1.0, 0.0)
    s = (m - 1.0) / (m + 1.0)
    s2 = s * s
    lnm = s * (2.0 + s2 * (2.0 / 3.0 + s2 * (2.0 / 5.0 + s2 * (2.0 / 7.0))))
    lse = ef * 0.6931471805599453 + lnm
    adj = sc_v[:] - lse
    st_v[pl.ds(0, _LANES)] = st_v[pl.ds(0, _LANES)] + adj
    st_v[pl.ds(_LANES, _LANES)] = st_v[pl.ds(_LANES, _LANES)] + adj

    pltpu.sync_copy(st_v, tv_hbm.at[wid])
    pltpu.sync_copy(sti_v, ti_hbm.at[wid])


@functools.cache
def _sc_topk():
    return pl.kernel(
        _sc_body,
        out_type=(
            jax.ShapeDtypeStruct((_BEAM, 2 * _LANES), jnp.float32),
            jax.ShapeDtypeStruct((_BEAM, 2 * _LANES), jnp.int32),
        ),
        mesh=plsc.VectorSubcoreMesh(core_axis_name="c", subcore_axis_name="s"),
        scratch_types=[
            pltpu.VMEM((_VOCAB,), jnp.float32),
            pltpu.VMEM((_CAP,), jnp.int32),
            pltpu.VMEM((2 * _LANES,), jnp.float32),
            pltpu.VMEM((2 * _LANES,), jnp.int32),
            pltpu.VMEM((_LANES,), jnp.float32),
        ],
        compiler_params=pltpu.CompilerParams(needs_layout_passes=False),
    )


def _tc_body(tv_ref, ti_ref, gs_ref, sm_ref,
             ng_ref, fs_ref, sl_ref, comb_ref, bi_ref):
    comb_ref[:] = tv_ref[:]
    rowi = lax.broadcasted_iota(jnp.int32, (_BEAM, _BEAM), 0)
    coli = lax.broadcasted_iota(jnp.int32, (_BEAM, _BEAM), 1)
    flat = rowi * _BEAM + coli

    def body(k, carry):
        cur = comb_ref[:]
        mval = jnp.max(cur)
        idx = jnp.min(jnp.where(cur == mval, flat, 1 << 20))
        r = idx // _BEAM
        c = idx - r * _BEAM
        fs_ref[pl.ds(k, 1), :] = jnp.broadcast_to(mval, (1, 1))
        ng_ref[pl.ds(k, 1), :] = gs_ref[pl.ds(r, 1), :]
        tirow = ti_ref[pl.ds(r, 1), :]
        ci = lax.broadcasted_iota(jnp.int32, (1, _BEAM), 1)
        tval = jnp.sum(jnp.where(ci == c, tirow, 0))
        bi_ref[pl.ds(k, 1), :] = jnp.broadcast_to(tval, (1, 1))
        comb_ref[:] = jnp.where(flat == idx, -jnp.inf, cur)
        return carry

    lax.fori_loop(0, _BEAM, body, 0)

    colm = lax.broadcasted_iota(jnp.int32, (_BEAM, _MAXLEN), 1)
    ng = ng_ref[:]
    ng2 = jnp.where(sm_ref[:] != 0, bi_ref[:], ng)
    ng_ref[:] = ng2
    sl_ref[:] = jnp.min(
        jnp.where(ng2 == _EOS, colm + 1, _MAXLEN), axis=1, keepdims=True)


def _build_tc(interpret=False):
    return pl.pallas_call(
        _tc_body,
        out_shape=(
            jax.ShapeDtypeStruct((_BEAM, _MAXLEN), jnp.int32),
            jax.ShapeDtypeStruct((_BEAM, 1), jnp.float32),
            jax.ShapeDtypeStruct((_BEAM, 1), jnp.int32),
        ),
        scratch_shapes=[
            pltpu.VMEM((_BEAM, _BEAM), jnp.float32),
            pltpu.VMEM((_BEAM, 1), jnp.int32),
        ],
        interpret=interpret,
    )


_tc_combine = _build_tc()


def kernel(logits, scores, gen_seq, step):
    scores_rep = jnp.broadcast_to(scores.reshape(_BEAM, 1), (_BEAM, _LANES))
    tv, ti = _sc_topk()(logits, scores_rep)
    stepmask = (
        lax.broadcasted_iota(jnp.int32, (1, _MAXLEN), 1)
        == jnp.asarray(step, jnp.int32)
    ).astype(jnp.int32)
    ng, fs, sl = _tc_combine(
        tv, ti,
        gen_seq.astype(jnp.int32),
        stepmask,
    )
    return ng.astype(gen_seq.dtype), fs.reshape(_BEAM), sl.reshape(_BEAM)


# speedup vs baseline: 1.0063x; 1.0063x over previous
import functools

import jax
import jax.numpy as jnp
from jax import lax
from jax.experimental import pallas as pl
from jax.experimental.pallas import tpu as pltpu
from jax.experimental.pallas import tpu_sc as plsc

_BEAM = 32
_VOCAB = 100000
_MAXLEN = 200
_EOS = 2
_LANES = 16
_NCHUNK = _VOCAB // _LANES
_NC = 2
_NS = 16


_CAPL = 512
_CAP = _CAPL * _LANES
_U1 = 10
_U2 = 10


def _merge_topk(st_v, sti_v, v, idx):
    sv, si = plsc.sort_key_val(v, idx, descending=True)
    hi = st_v[pl.ds(0, _LANES)]
    hii = sti_v[pl.ds(0, _LANES)]
    lo = st_v[pl.ds(_LANES, _LANES)]
    loi = sti_v[pl.ds(_LANES, _LANES)]
    rv = lax.rev(sv, (0,))
    ri = lax.rev(si, (0,))
    m = lo >= rv
    u = jnp.where(m, lo, rv)
    ui = jnp.where(m, loi, ri)
    us, usi = plsc.sort_key_val(u, ui, descending=True)
    ru = lax.rev(us, (0,))
    rui = lax.rev(usi, (0,))
    m2 = hi >= ru
    a = jnp.where(m2, hi, ru)
    ai = jnp.where(m2, hii, rui)
    b = jnp.where(m2, ru, hi)
    bi = jnp.where(m2, rui, hii)
    hs, hsi = plsc.sort_key_val(a, ai, descending=True)
    ls, lsi = plsc.sort_key_val(b, bi, descending=True)
    st_v[pl.ds(0, _LANES)] = hs
    sti_v[pl.ds(0, _LANES)] = hsi
    st_v[pl.ds(_LANES, _LANES)] = ls
    sti_v[pl.ds(_LANES, _LANES)] = lsi


def _sc_body(logits_hbm, scores_hbm, tv_hbm, ti_hbm,
             row_v, cand_v, st_v, sti_v, sc_v):
    wid = lax.axis_index("s") * _NC + lax.axis_index("c")
    pltpu.sync_copy(logits_hbm.at[wid], row_v)

    neg = jnp.full((_LANES,), -jnp.inf, jnp.float32)

    @plsc.parallel_loop(
        0, _VOCAB, _LANES, unroll=_U1,
        carry=(neg, neg, jnp.zeros((_LANES,), jnp.float32)))
    def p1(i, carry):
        m1, m2, acc = carry
        v = row_v[pl.ds(i, _LANES)]
        acc = acc + jnp.exp(v)
        m2 = jnp.maximum(m2, jnp.minimum(m1, v))
        m1 = jnp.maximum(m1, v)
        return m1, m2, acc

    m1, m2, acc = p1
    thr = jnp.broadcast_to(jnp.min(m2), (_LANES,))

    lane_base = lax.iota(jnp.int32, _LANES) * _CAPL

    @plsc.parallel_loop(
        0, _VOCAB, _LANES, unroll=_U2,
        carry=jnp.zeros((_LANES,), jnp.int32))
    def p2(i, cnt):
        v = row_v[pl.ds(i, _LANES)]
        bvec = lax.iota(jnp.int32, _LANES) + i
        msk = v >= thr
        mok = jnp.logical_and(msk, cnt < _CAPL)
        plsc.store_scatter(cand_v, [lane_base + cnt], bvec, mask=mok)
        return cnt + jnp.where(msk, 1, 0)

    cnt = p2

    st_v[pl.ds(0, _LANES)] = neg
    st_v[pl.ds(_LANES, _LANES)] = neg
    zero_i = jnp.zeros((_LANES,), jnp.int32)
    sti_v[pl.ds(0, _LANES)] = zero_i
    sti_v[pl.ds(_LANES, _LANES)] = zero_i

    cnt = jnp.minimum(cnt, _CAPL)
    for lane in range(_LANES):
        cl = cnt[lane]

        def p3(j, carry, lane=lane, cl=cl):
            b0 = lane * _CAPL + j * _LANES
            valid = lax.iota(jnp.int32, _LANES) + j * _LANES < cl
            idx = jnp.where(valid, cand_v[pl.ds(b0, _LANES)], 0)
            v = plsc.load_gather(row_v, [idx])
            v = jnp.where(valid, v, -jnp.inf)
            _merge_topk(st_v, sti_v, v, idx)
            return carry

        lax.fori_loop(0, (cl + _LANES - 1) // _LANES, p3, 0)

    pltpu.sync_copy(scores_hbm.at[wid], sc_v)
    x = jnp.broadcast_to(jnp.sum(acc), (_LANES,))
    bits = plsc.bitcast(x, jnp.int32)
    e = jnp.right_shift(bits, 23) - 127
    m = plsc.bitcast((bits & 0x7FFFFF) | 0x3F800000, jnp.float32)
    half = m > 1.4142135
    m = jnp.where(half, m * 0.5, m)
    ef = e.astype(jnp.float32) + jnp.where(half, 1.0, 0.0)
    s = (m - 1.0) / (m + 1.0)
    s2 = s * s
    lnm = s * (2.0 + s2 * (2.0 / 3.0 + s2 * (2.0 / 5.0 + s2 * (2.0 / 7.0))))
    lse = ef * 0.6931471805599453 + lnm
    adj = sc_v[:] - lse
    st_v[pl.ds(0, _LANES)] = st_v[pl.ds(0, _LANES)] + adj
    st_v[pl.ds(_LANES, _LANES)] = st_v[pl.ds(_LANES, _LANES)] + adj

    pltpu.sync_copy(st_v, tv_hbm.at[wid])
    pltpu.sync_copy(sti_v, ti_hbm.at[wid])


@functools.cache
def _sc_topk():
    return pl.kernel(
        _sc_body,
        out_type=(
            jax.ShapeDtypeStruct((_BEAM, 2 * _LANES), jnp.float32),
            jax.ShapeDtypeStruct((_BEAM, 2 * _LANES), jnp.int32),
        ),
        mesh=plsc.VectorSubcoreMesh(core_axis_name="c", subcore_axis_name="s"),
        scratch_types=[
            pltpu.VMEM((_VOCAB,), jnp.float32),
            pltpu.VMEM((_CAP,), jnp.int32),
            pltpu.VMEM((2 * _LANES,), jnp.float32),
            pltpu.VMEM((2 * _LANES,), jnp.int32),
            pltpu.VMEM((_LANES,), jnp.float32),
        ],
        compiler_params=pltpu.CompilerParams(needs_layout_passes=False),
    )


def _tc_body(tv_ref, ti_ref, gs_ref, sm_ref, ng_ref, fs_ref, sl_ref):
    rowi = lax.broadcasted_iota(jnp.int32, (_BEAM, _BEAM), 0)
    coli = lax.broadcasted_iota(jnp.int32, (_BEAM, _BEAM), 1)
    flat = rowi * _BEAM + coli
    kio = lax.broadcasted_iota(jnp.int32, (_BEAM, 1), 0)
    ti_f = ti_ref[:].astype(jnp.float32)
    zc = jnp.zeros((_BEAM, 1), jnp.float32)

    def body(k, carry):
        comb, fs, bi, rv = carry
        mval = jnp.max(comb)
        idx = jnp.min(jnp.where(comb == mval, flat, 1 << 20))
        sel = flat == idx
        tval = jnp.sum(jnp.where(sel, ti_f, 0.0))
        at_k = kio == k
        fs = jnp.where(at_k, mval, fs)
        bi = jnp.where(at_k, tval, bi)
        rv = jnp.where(at_k, idx // _BEAM, rv)
        comb = jnp.where(sel, -jnp.inf, comb)
        return comb, fs, bi, rv

    _, fs, bi, rv = lax.fori_loop(
        0, _BEAM, body,
        (tv_ref[:], zc, zc, jnp.zeros((_BEAM, 1), jnp.int32)))

    onehot = (coli == rv).astype(jnp.float32)
    ng = jax.lax.dot(onehot, gs_ref[:].astype(jnp.float32),
                     precision=jax.lax.Precision.HIGHEST)
    ng = jnp.where(sm_ref[:] != 0, bi, ng)
    ng_i = ng.astype(jnp.int32)
    ng_ref[:] = ng_i
    colm = lax.broadcasted_iota(jnp.int32, (_BEAM, _MAXLEN), 1)
    sl_ref[:] = jnp.min(
        jnp.where(ng_i == _EOS, colm + 1, _MAXLEN), axis=1, keepdims=True)
    fs_ref[:] = fs


def _build_tc(interpret=False):
    return pl.pallas_call(
        _tc_body,
        out_shape=(
            jax.ShapeDtypeStruct((_BEAM, _MAXLEN), jnp.int32),
            jax.ShapeDtypeStruct((_BEAM, 1), jnp.float32),
            jax.ShapeDtypeStruct((_BEAM, 1), jnp.int32),
        ),
        interpret=interpret,
    )


_tc_combine = _build_tc()


def kernel(logits, scores, gen_seq, step):
    scores_rep = jnp.broadcast_to(scores.reshape(_BEAM, 1), (_BEAM, _LANES))
    tv, ti = _sc_topk()(logits, scores_rep)
    stepmask = (
        lax.broadcasted_iota(jnp.int32, (1, _MAXLEN), 1)
        == jnp.asarray(step, jnp.int32)
    ).astype(jnp.int32)
    ng, fs, sl = _tc_combine(
        tv, ti,
        gen_seq.astype(jnp.int32),
        stepmask,
    )
    return ng.astype(gen_seq.dtype), fs.reshape(_BEAM), sl.reshape(_BEAM)


# speedup vs baseline: 1.0567x; 1.0501x over previous
import functools

import jax
import jax.numpy as jnp
from jax import lax
from jax.experimental import pallas as pl
from jax.experimental.pallas import tpu as pltpu
from jax.experimental.pallas import tpu_sc as plsc

_BEAM = 32
_VOCAB = 100000
_MAXLEN = 200
_EOS = 2
_LANES = 16
_NCHUNK = _VOCAB // _LANES
_NC = 2
_NS = 16


_CAPL = 512
_CAP = _CAPL * _LANES
_U1 = 10
_U2 = 10


def _merge_topk(st_v, sti_v, v, idx):
    sv, si = plsc.sort_key_val(v, idx, descending=True)
    hi = st_v[pl.ds(0, _LANES)]
    hii = sti_v[pl.ds(0, _LANES)]
    lo = st_v[pl.ds(_LANES, _LANES)]
    loi = sti_v[pl.ds(_LANES, _LANES)]
    rv = lax.rev(sv, (0,))
    ri = lax.rev(si, (0,))
    m = lo >= rv
    u = jnp.where(m, lo, rv)
    ui = jnp.where(m, loi, ri)
    us, usi = plsc.sort_key_val(u, ui, descending=True)
    ru = lax.rev(us, (0,))
    rui = lax.rev(usi, (0,))
    m2 = hi >= ru
    a = jnp.where(m2, hi, ru)
    ai = jnp.where(m2, hii, rui)
    b = jnp.where(m2, ru, hi)
    bi = jnp.where(m2, rui, hii)
    hs, hsi = plsc.sort_key_val(a, ai, descending=True)
    ls, lsi = plsc.sort_key_val(b, bi, descending=True)
    st_v[pl.ds(0, _LANES)] = hs
    sti_v[pl.ds(0, _LANES)] = hsi
    st_v[pl.ds(_LANES, _LANES)] = ls
    sti_v[pl.ds(_LANES, _LANES)] = lsi


def _sc_body(logits_hbm, scores_hbm, tv_hbm, ti_hbm,
             row_v, cand_v, st_v, sti_v, sc_v):
    wid = lax.axis_index("s") * _NC + lax.axis_index("c")
    pltpu.sync_copy(logits_hbm.at[wid], row_v)

    neg = jnp.full((_LANES,), -jnp.inf, jnp.float32)

    @plsc.parallel_loop(
        0, _VOCAB, _LANES, unroll=_U1,
        carry=(neg, neg, jnp.zeros((_LANES,), jnp.float32)))
    def p1(i, carry):
        m1, m2, acc = carry
        v = row_v[pl.ds(i, _LANES)]
        acc = acc + jnp.exp(v)
        m2 = jnp.maximum(m2, jnp.minimum(m1, v))
        m1 = jnp.maximum(m1, v)
        return m1, m2, acc

    m1, m2, acc = p1
    thr = jnp.broadcast_to(jnp.min(m2), (_LANES,))

    lane_base = lax.iota(jnp.int32, _LANES) * _CAPL

    @plsc.parallel_loop(
        0, _VOCAB, _LANES, unroll=_U2,
        carry=jnp.zeros((_LANES,), jnp.int32))
    def p2(i, cnt):
        v = row_v[pl.ds(i, _LANES)]
        bvec = lax.iota(jnp.int32, _LANES) + i
        msk = v >= thr
        mok = jnp.logical_and(msk, cnt < _CAPL)
        plsc.store_scatter(cand_v, [lane_base + cnt], bvec, mask=mok)
        return cnt + jnp.where(msk, 1, 0)

    cnt = p2

    st_v[pl.ds(0, _LANES)] = neg
    st_v[pl.ds(_LANES, _LANES)] = neg
    zero_i = jnp.zeros((_LANES,), jnp.int32)
    sti_v[pl.ds(0, _LANES)] = zero_i
    sti_v[pl.ds(_LANES, _LANES)] = zero_i

    cnt = jnp.minimum(cnt, _CAPL)
    for lane in range(_LANES):
        cl = cnt[lane]

        def p3(j, carry, lane=lane, cl=cl):
            b0 = lane * _CAPL + j * _LANES
            valid = lax.iota(jnp.int32, _LANES) + j * _LANES < cl
            idx = jnp.where(valid, cand_v[pl.ds(b0, _LANES)], 0)
            v = plsc.load_gather(row_v, [idx])
            v = jnp.where(valid, v, -jnp.inf)
            _merge_topk(st_v, sti_v, v, idx)
            return carry

        lax.fori_loop(0, (cl + _LANES - 1) // _LANES, p3, 0)

    base16 = (wid // _LANES) * _LANES
    pltpu.sync_copy(scores_hbm.at[pl.ds(base16, _LANES)], sc_v)
    x = jnp.broadcast_to(jnp.sum(acc), (_LANES,))
    bits = plsc.bitcast(x, jnp.int32)
    e = jnp.right_shift(bits, 23) - 127
    m = plsc.bitcast((bits & 0x7FFFFF) | 0x3F800000, jnp.float32)
    half = m > 1.4142135
    m = jnp.where(half, m * 0.5, m)
    ef = e.astype(jnp.float32) + jnp.where(half, 1.0, 0.0)
    s = (m - 1.0) / (m + 1.0)
    s2 = s * s
    lnm = s * (2.0 + s2 * (2.0 / 3.0 + s2 * (2.0 / 5.0 + s2 * (2.0 / 7.0))))
    lse = ef * 0.6931471805599453 + lnm
    score = plsc.load_gather(
        sc_v, [jnp.broadcast_to(wid - base16, (_LANES,))])
    adj = score - lse
    st_v[pl.ds(0, _LANES)] = st_v[pl.ds(0, _LANES)] + adj
    st_v[pl.ds(_LANES, _LANES)] = st_v[pl.ds(_LANES, _LANES)] + adj

    pltpu.sync_copy(st_v, tv_hbm.at[wid])
    pltpu.sync_copy(sti_v, ti_hbm.at[wid])


@functools.cache
def _sc_topk():
    return pl.kernel(
        _sc_body,
        out_type=(
            jax.ShapeDtypeStruct((_BEAM, 2 * _LANES), jnp.float32),
            jax.ShapeDtypeStruct((_BEAM, 2 * _LANES), jnp.int32),
        ),
        mesh=plsc.VectorSubcoreMesh(core_axis_name="c", subcore_axis_name="s"),
        scratch_types=[
            pltpu.VMEM((_VOCAB,), jnp.float32),
            pltpu.VMEM((_CAP,), jnp.int32),
            pltpu.VMEM((2 * _LANES,), jnp.float32),
            pltpu.VMEM((2 * _LANES,), jnp.int32),
            pltpu.VMEM((_LANES,), jnp.float32),
        ],
        compiler_params=pltpu.CompilerParams(needs_layout_passes=False),
    )


def _tc_body(step_ref, tv_ref, ti_ref, gs_ref, ng_ref, fs_ref, sl_ref):
    rowi = lax.broadcasted_iota(jnp.int32, (_BEAM, _BEAM), 0)
    coli = lax.broadcasted_iota(jnp.int32, (_BEAM, _BEAM), 1)
    flat = rowi * _BEAM + coli
    kio_r = lax.broadcasted_iota(jnp.int32, (1, _BEAM), 1)
    kio_c = lax.broadcasted_iota(jnp.int32, (_BEAM, 1), 0)
    ti_f = ti_ref[:].astype(jnp.float32)

    def body(k, carry):
        comb, fs, bi, rv = carry
        mval = jnp.max(comb)
        idx = jnp.min(jnp.where(comb == mval, flat, 1 << 20))
        sel = flat == idx
        tval = jnp.sum(jnp.where(sel, ti_f, 0.0))
        fs = jnp.where(kio_r == k, mval, fs)
        bi = jnp.where(kio_c == k, tval, bi)
        rv = jnp.where(kio_c == k, idx // _BEAM, rv)
        comb = jnp.where(sel, -jnp.inf, comb)
        return comb, fs, bi, rv

    _, fs, bi, rv = lax.fori_loop(
        0, _BEAM, body,
        (tv_ref[:], jnp.zeros((1, _BEAM), jnp.float32),
         jnp.zeros((_BEAM, 1), jnp.float32),
         jnp.zeros((_BEAM, 1), jnp.int32)))

    onehot = (coli == rv).astype(jnp.float32)
    ng = jax.lax.dot(onehot, gs_ref[:].astype(jnp.float32),
                     precision=jax.lax.Precision.HIGHEST)
    colm = lax.broadcasted_iota(jnp.int32, (_BEAM, _MAXLEN), 1)
    ng = jnp.where(colm == step_ref[0, 0], bi, ng)
    ng_i = ng.astype(jnp.int32)
    ng_ref[:] = ng_i
    lens = jnp.min(
        jnp.where(ng_i == _EOS, colm + 1, _MAXLEN), axis=1, keepdims=True)
    diag = jnp.where(rowi == coli, lens.astype(jnp.float32), 0.0)
    lens_row = jax.lax.dot(jnp.ones((1, _BEAM), jnp.float32), diag,
                           precision=jax.lax.Precision.HIGHEST)
    sl_ref[:] = lens_row.astype(jnp.int32).reshape(_BEAM)
    fs_ref[:] = fs.reshape(_BEAM)


def _build_tc(interpret=False):
    return pl.pallas_call(
        _tc_body,
        in_specs=[
            pl.BlockSpec(memory_space=pltpu.SMEM),
            pl.BlockSpec(),
            pl.BlockSpec(),
            pl.BlockSpec(),
        ],
        out_shape=(
            jax.ShapeDtypeStruct((_BEAM, _MAXLEN), jnp.int32),
            jax.ShapeDtypeStruct((_BEAM,), jnp.float32),
            jax.ShapeDtypeStruct((_BEAM,), jnp.int32),
        ),
        interpret=interpret,
    )


_tc_combine = _build_tc()


def kernel(logits, scores, gen_seq, step):
    tv, ti = _sc_topk()(logits, scores)
    step_arr = jnp.asarray(step, jnp.int32).reshape(1, 1)
    return _tc_combine(step_arr, tv, ti, gen_seq)
